# baseline (speedup 1.0000x reference)
import functools

import jax
import jax.numpy as jnp
from jax import lax
from jax.experimental import pallas as pl
from jax.experimental.pallas import tpu as pltpu
from jax.experimental.pallas import tpu_sc as plsc

E = 320000
V = 10000
D = 128
NC = 2
NS = 16
NW = NC * NS
EW = E // NW
CH = 128
NFULL = EW // CH
TAIL = EW - NFULL * CH
NG = CH // 16


def _decode_body(zc, ei, out, idxu, idxi, ru0, ri0, ru1, ri1,
                 rut, rit, outv, dots, su0, si0, su1, si1, sut, sit):
    wid = lax.axis_index("s") * NC + lax.axis_index("c")
    base = wid * EW
    col0 = lax.iota(jnp.int32, 16) * 16

    pltpu.sync_copy(ei.at[0, pl.ds(base, EW)], idxu)
    pltpu.sync_copy(ei.at[1, pl.ds(base, EW)], idxi)

    def _offs(t, carry):
        idxi[pl.ds(t * 16, 16)] = idxi[pl.ds(t * 16, 16)] + V
        return carry
    lax.fori_loop(0, EW // 16, _offs, 0)

    def _issue(c, bu, bi, su, si):
        pltpu.async_copy(zc.at[idxu.at[pl.ds(c * CH, CH)]], bu, su)
        pltpu.async_copy(zc.at[idxi.at[pl.ds(c * CH, CH)]], bi, si)

    def _wait(bu, bi, su, si):
        pltpu.make_async_copy(zc.at[pl.ds(0, bu.shape[0])], bu, su).wait()
        pltpu.make_async_copy(zc.at[pl.ds(0, bi.shape[0])], bi, si).wait()

    def _loads(bu, bi, e):
        return (bu[e, pl.ds(0, 32)], bi[e, pl.ds(0, 32)],
                bu[e, pl.ds(32, 32)], bi[e, pl.ds(32, 32)],
                bu[e, pl.ds(64, 32)], bi[e, pl.ds(64, 32)],
                bu[e, pl.ds(96, 32)], bi[e, pl.ds(96, 32)])

    def _compute(c, bu, bi, n):
        def _group16(g, carry):
            e0 = g * 16
            r = _loads(bu, bi, e0)
            for j in range(16):
                nxt = _loads(bu, bi, e0 + j + 1) if j < 15 else None
                acc32 = (r[0] * r[1] + r[2] * r[3]) + (r[4] * r[5] + r[6] * r[7])
                pa, pb = plsc.unpack(acc32, format=plsc.PackFormat.INTERLEAVED)
                dots[pl.ds(g * 256 + j * 16, 16)] = pa + pb
                r = nxt
            return carry
        lax.fori_loop(0, n // 16, _group16, 0)

        def _reduce(g, carry):
            b0 = g * 256
            cols = [plsc.load_gather(dots, [b0 + col0 + l]) for l in range(16)]
            while len(cols) > 1:
                cols = [cols[i] + cols[i + 1] for i in range(0, len(cols), 2)]
            outv[pl.ds(c * CH + g * 16, 16)] = 1.0 / (1.0 + jnp.exp(-cols[0]))
            return carry
        lax.fori_loop(0, n // 16, _reduce, 0)

    _issue(0, ru0, ri0, su0, si0)
    _issue(1, ru1, ri1, su1, si1)

    def _pair(tt, carry):
        c0 = tt * 2
        _wait(ru0, ri0, su0, si0)
        _compute(c0, ru0, ri0, CH)
        _issue(c0 + 2, ru0, ri0, su0, si0)
        _wait(ru1, ri1, su1, si1)
        _compute(c0 + 1, ru1, ri1, CH)
        _issue(c0 + 3, ru1, ri1, su1, si1)
        return carry

    lax.fori_loop(0, NFULL // 2 - 1, _pair, 0)

    _wait(ru0, ri0, su0, si0)
    _compute(NFULL - 2, ru0, ri0, CH)
    pltpu.async_copy(zc.at[idxu.at[pl.ds(NFULL * CH, TAIL)]], rut, sut)
    pltpu.async_copy(zc.at[idxi.at[pl.ds(NFULL * CH, TAIL)]], rit, sit)
    _wait(ru1, ri1, su1, si1)
    _compute(NFULL - 1, ru1, ri1, CH)
    _wait(rut, rit, sut, sit)
    _compute(NFULL, rut, rit, TAIL)

    pltpu.sync_copy(outv, out.at[pl.ds(base, EW)])


_decode = functools.partial(
    pl.kernel,
    mesh=plsc.VectorSubcoreMesh(core_axis_name="c", subcore_axis_name="s"),
    out_type=jax.ShapeDtypeStruct((E,), jnp.float32),
    compiler_params=pltpu.CompilerParams(needs_layout_passes=False,
                                        use_tc_tiling_on_sc=False),
    scratch_types=[
        pltpu.VMEM((EW,), jnp.int32),
        pltpu.VMEM((EW,), jnp.int32),
        pltpu.VMEM((CH, D), jnp.bfloat16),
        pltpu.VMEM((CH, D), jnp.bfloat16),
        pltpu.VMEM((CH, D), jnp.bfloat16),
        pltpu.VMEM((CH, D), jnp.bfloat16),
        pltpu.VMEM((TAIL, D), jnp.bfloat16),
        pltpu.VMEM((TAIL, D), jnp.bfloat16),
        pltpu.VMEM((EW,), jnp.float32),
        pltpu.VMEM((CH * 16,), jnp.float32),
        pltpu.SemaphoreType.DMA,
        pltpu.SemaphoreType.DMA,
        pltpu.SemaphoreType.DMA,
        pltpu.SemaphoreType.DMA,
        pltpu.SemaphoreType.DMA,
        pltpu.SemaphoreType.DMA,
    ],
)(_decode_body)


def _cast_body(a_ref, b_ref, o_ref):
    i = pl.program_id(0)

    @pl.when(i < 10)
    def _():
        o_ref[...] = a_ref[...].astype(jnp.bfloat16)

    @pl.when(i >= 10)
    def _():
        o_ref[...] = b_ref[...].astype(jnp.bfloat16)


_cast = pl.pallas_call(
    _cast_body,
    grid=(20,),
    in_specs=[pl.BlockSpec((1000, 128), lambda i: (jnp.minimum(i, 9), 0)),
              pl.BlockSpec((1000, 128), lambda i: (jnp.maximum(i - 10, 0), 0))],
    out_specs=pl.BlockSpec((1000, 128), lambda i: (i, 0)),
    out_shape=jax.ShapeDtypeStruct((2 * V, D), jnp.bfloat16),
)


def kernel(z_user, z_item, edge_index):
    zcat = _cast(z_user, z_item)
    return _decode(zcat, edge_index.astype(jnp.int32))


# speedup vs baseline: 1.1189x; 1.1189x over previous
import functools

import jax
import jax.numpy as jnp
from jax import lax
from jax.experimental import pallas as pl
from jax.experimental.pallas import tpu as pltpu
from jax.experimental.pallas import tpu_sc as plsc

E = 320000
D = 128
NC = 2
NS = 16
NW = NC * NS
EW = E // NW
CH = 128
NFULL = EW // CH
TAIL = EW - NFULL * CH
NG = CH // 16


def _decode_body(zu, zi, ei, out, idxu, idxi, ru0, ri0, ru1, ri1,
                 rut, rit, outv, dots, su0, si0, su1, si1, sut, sit):
    wid = lax.axis_index("s") * NC + lax.axis_index("c")
    base = wid * EW
    col0 = lax.iota(jnp.int32, 16) * 16

    pltpu.sync_copy(ei.at[0, pl.ds(base, EW)], idxu)
    pltpu.sync_copy(ei.at[1, pl.ds(base, EW)], idxi)

    def _issue(c, bu, bi, su, si):
        pltpu.async_copy(zu.at[idxu.at[pl.ds(c * CH, CH)]], bu, su)
        pltpu.async_copy(zi.at[idxi.at[pl.ds(c * CH, CH)]], bi, si)

    def _wait(bu, bi, su, si):
        pltpu.make_async_copy(zu.at[pl.ds(0, bu.shape[0])], bu, su).wait()
        pltpu.make_async_copy(zi.at[pl.ds(0, bi.shape[0])], bi, si).wait()

    def _loads(bu, bi, e):
        return (bu[e, pl.ds(0, 32)], bi[e, pl.ds(0, 32)],
                bu[e, pl.ds(32, 32)], bi[e, pl.ds(32, 32)],
                bu[e, pl.ds(64, 32)], bi[e, pl.ds(64, 32)],
                bu[e, pl.ds(96, 32)], bi[e, pl.ds(96, 32)])

    def _compute(c, bu, bi, n):
        def _group16(g, carry):
            e0 = g * 16
            r0 = _loads(bu, bi, e0)
            r1 = _loads(bu, bi, e0 + 1)
            for j in range(16):
                nxt = _loads(bu, bi, e0 + j + 2) if j < 14 else None
                acc32 = ((r0[0] * r0[1] + r0[2] * r0[3])
                         + (r0[4] * r0[5] + r0[6] * r0[7]))
                pa, pb = plsc.unpack(acc32, format=plsc.PackFormat.INTERLEAVED)
                dots[pl.ds(g * 256 + j * 16, 16)] = pa + pb
                r0, r1 = r1, nxt
            return carry
        lax.fori_loop(0, n // 16, _group16, 0)

        def _reduce(g, carry):
            b0 = g * 256
            cols = [plsc.load_gather(dots, [b0 + col0 + l]) for l in range(16)]
            while len(cols) > 1:
                cols = [cols[i] + cols[i + 1] for i in range(0, len(cols), 2)]
            outv[pl.ds(c * CH + g * 16, 16)] = 1.0 / (1.0 + jnp.exp(-cols[0]))
            return carry
        lax.fori_loop(0, n // 16, _reduce, 0)

    _issue(0, ru0, ri0, su0, si0)
    _issue(1, ru1, ri1, su1, si1)

    def _pair(tt, carry):
        c0 = tt * 2
        _wait(ru0, ri0, su0, si0)
        _compute(c0, ru0, ri0, CH)
        _issue(c0 + 2, ru0, ri0, su0, si0)
        _wait(ru1, ri1, su1, si1)
        _compute(c0 + 1, ru1, ri1, CH)
        _issue(c0 + 3, ru1, ri1, su1, si1)
        return carry

    lax.fori_loop(0, NFULL // 2 - 1, _pair, 0)

    _wait(ru0, ri0, su0, si0)
    _compute(NFULL - 2, ru0, ri0, CH)
    pltpu.async_copy(zu.at[idxu.at[pl.ds(NFULL * CH, TAIL)]], rut, sut)
    pltpu.async_copy(zi.at[idxi.at[pl.ds(NFULL * CH, TAIL)]], rit, sit)
    _wait(ru1, ri1, su1, si1)
    _compute(NFULL - 1, ru1, ri1, CH)
    _wait(rut, rit, sut, sit)
    _compute(NFULL, rut, rit, TAIL)

    pltpu.sync_copy(outv, out.at[pl.ds(base, EW)])


_decode = functools.partial(
    pl.kernel,
    mesh=plsc.VectorSubcoreMesh(core_axis_name="c", subcore_axis_name="s"),
    out_type=jax.ShapeDtypeStruct((E,), jnp.float32),
    compiler_params=pltpu.CompilerParams(needs_layout_passes=False,
                                        use_tc_tiling_on_sc=False),
    scratch_types=[
        pltpu.VMEM((EW,), jnp.int32),
        pltpu.VMEM((EW,), jnp.int32),
        pltpu.VMEM((CH, D), jnp.bfloat16),
        pltpu.VMEM((CH, D), jnp.bfloat16),
        pltpu.VMEM((CH, D), jnp.bfloat16),
        pltpu.VMEM((CH, D), jnp.bfloat16),
        pltpu.VMEM((TAIL, D), jnp.bfloat16),
        pltpu.VMEM((TAIL, D), jnp.bfloat16),
        pltpu.VMEM((EW,), jnp.float32),
        pltpu.VMEM((CH * 16,), jnp.float32),
        pltpu.SemaphoreType.DMA,
        pltpu.SemaphoreType.DMA,
        pltpu.SemaphoreType.DMA,
        pltpu.SemaphoreType.DMA,
        pltpu.SemaphoreType.DMA,
        pltpu.SemaphoreType.DMA,
    ],
)(_decode_body)


def _cast_body(a_ref, b_ref, oa_ref, ob_ref):
    oa_ref[...] = a_ref[...].astype(jnp.bfloat16)
    ob_ref[...] = b_ref[...].astype(jnp.bfloat16)


_cast = pl.pallas_call(
    _cast_body,
    grid=(10,),
    in_specs=[pl.BlockSpec((1000, 128), lambda i: (i, 0)),
              pl.BlockSpec((1000, 128), lambda i: (i, 0))],
    out_specs=[pl.BlockSpec((1000, 128), lambda i: (i, 0)),
               pl.BlockSpec((1000, 128), lambda i: (i, 0))],
    out_shape=[jax.ShapeDtypeStruct((10000, 128), jnp.bfloat16)] * 2,
)


def kernel(z_user, z_item, edge_index):
    zu_bf, zi_bf = _cast(z_user, z_item)
    return _decode(zu_bf, zi_bf, edge_index.astype(jnp.int32))
